# baseline (speedup 1.0000x reference)
import functools

import jax
import jax.numpy as jnp
from jax import lax
from jax.experimental import pallas as pl
from jax.experimental.pallas import tpu as pltpu
from jax.experimental.pallas import tpu_sc as plsc

_V = 1_000_000
_DIM = 64
_PAD = 128
_NW = 32
_CHUNK = 128
_K = 2
_NBUF = 2
_ABLK = 4096
_CBLK = 2048


def _pad_table(wt):
    def body(i_ref, o_ref):
        o_ref[:, :_DIM] = i_ref[...].T

    return pl.pallas_call(
        body,
        grid=(pl.cdiv(_V, _ABLK),),
        in_specs=[pl.BlockSpec((_DIM, _ABLK), lambda j: (0, j))],
        out_specs=pl.BlockSpec((_ABLK, _PAD), lambda j: (j, 0)),
        out_shape=jax.ShapeDtypeStruct((_V, _PAD), jnp.float32),
    )(wt)


def _untranspose(g, b, s):
    def body(i_ref, o_ref):
        o_ref[...] = i_ref[:, :_DIM].T[None]

    nb = b // _CBLK
    return pl.pallas_call(
        body,
        grid=(s, nb),
        in_specs=[pl.BlockSpec((_CBLK, _PAD), lambda si, j: (si * nb + j, 0))],
        out_specs=pl.BlockSpec((1, _DIM, _CBLK), lambda si, j: (si, 0, j)),
        out_shape=jax.ShapeDtypeStruct((s, _DIM, b), jnp.float32),
    )(g)


def _gather_sc(idx, table, *, ngroup):
    nchunk = ngroup * _K
    rows = _K * _CHUNK
    mesh = plsc.VectorSubcoreMesh(
        core_axis_name="c", subcore_axis_name="s", num_cores=2, num_subcores=16
    )

    @functools.partial(
        pl.kernel,
        out_type=jax.ShapeDtypeStruct((_NW * nchunk * _CHUNK, _PAD), jnp.float32),
        mesh=mesh,
        scratch_types=[
            pltpu.VMEM((nchunk, _CHUNK), jnp.int32),
            pltpu.VMEM((_NBUF, rows, _PAD), jnp.float32),
            pltpu.SemaphoreType.DMA,
            pltpu.SemaphoreType.DMA,
        ],
        compiler_params=pltpu.CompilerParams(use_tc_tiling_on_sc=False),
    )
    def body(idx_hbm, table_hbm, out_hbm, idx_v, rows_v, gsem0, gsem1):
        cid = lax.axis_index("c")
        sid = lax.axis_index("s")
        wid = sid * 2 + cid
        base = wid * nchunk * _CHUNK
        pltpu.sync_copy(idx_hbm.at[wid], idx_v)
        sems = (gsem0, gsem1)

        def gather_group(g, p, sem):
            for q in range(_K):
                pltpu.async_copy(
                    table_hbm.at[idx_v.at[g * _K + q]],
                    rows_v.at[p].at[pl.ds(q * _CHUNK, _CHUNK)],
                    sem,
                )

        def drain_group(g, p, sem):
            for q in range(_K):
                pltpu.make_async_copy(
                    table_hbm.at[idx_v.at[g * _K + q]],
                    rows_v.at[p].at[pl.ds(q * _CHUNK, _CHUNK)],
                    sem,
                ).wait()

        for p in range(_NBUF):
            gather_group(p, p, sems[p])

        @pl.loop(0, ngroup, step=_NBUF)
        def _(g):
            for p in range(_NBUF):
                cur = g + p
                drain_group(cur, p, sems[p])
                pltpu.sync_copy(
                    rows_v.at[p], out_hbm.at[pl.ds(base + cur * rows, rows)]
                )
                nxt = cur + _NBUF

                @pl.when(nxt < ngroup)
                def _():
                    gather_group(nxt, p, sems[p])

    return body(idx, table)


@jax.jit
def _embed(token_ids, weight):
    b, s = token_ids.shape
    nchunk = b * s // _NW // _CHUNK
    table = _pad_table(weight.T)
    idx = token_ids.T.reshape(_NW, nchunk, _CHUNK).astype(jnp.int32)
    g = _gather_sc(idx, table, ngroup=nchunk // _K)
    return _untranspose(g, b, s).transpose(2, 0, 1)


def kernel(token_ids, weight):
    return _embed(token_ids, weight)


# speedup vs baseline: 1.0840x; 1.0840x over previous
import functools

import jax
import jax.numpy as jnp
from jax import lax
from jax.experimental import pallas as pl
from jax.experimental.pallas import tpu as pltpu
from jax.experimental.pallas import tpu_sc as plsc

_V = 1_000_000
_DIM = 64
_PAD = 128
_NW = 32
_CHUNK = 128
_K = 2
_NBUF = 2
_ABLK = 4096
_CBLK = 2048


def _pad_table(wt):
    def body(i_ref, o_ref):
        o_ref[:, :_DIM] = i_ref[...].T

    return pl.pallas_call(
        body,
        grid=(pl.cdiv(_V, _ABLK),),
        in_specs=[pl.BlockSpec((_DIM, _ABLK), lambda j: (0, j))],
        out_specs=pl.BlockSpec((_ABLK, _PAD), lambda j: (j, 0)),
        out_shape=jax.ShapeDtypeStruct((_V, _PAD), jnp.float32),
    )(wt)


def _untranspose(g, b, s):
    def body(i_ref, o_ref):
        o_ref[...] = i_ref[:, :_DIM].T[None]

    nb = b // _CBLK
    return pl.pallas_call(
        body,
        grid=(s, nb),
        in_specs=[pl.BlockSpec((_CBLK, _PAD), lambda si, j: (si * nb + j, 0))],
        out_specs=pl.BlockSpec((1, _DIM, _CBLK), lambda si, j: (si, 0, j)),
        out_shape=jax.ShapeDtypeStruct((s, _DIM, b), jnp.float32),
    )(g)


def _gather_sc(idx, table, *, ngroup):
    nchunk = ngroup * _K
    rows = _K * _CHUNK
    mesh = plsc.VectorSubcoreMesh(
        core_axis_name="c", subcore_axis_name="s", num_cores=2, num_subcores=16
    )

    @functools.partial(
        pl.kernel,
        out_type=jax.ShapeDtypeStruct((_NW * nchunk * _CHUNK, _PAD), jnp.float32),
        mesh=mesh,
        scratch_types=[
            pltpu.VMEM((nchunk, _CHUNK), jnp.int32),
            pltpu.VMEM((_NBUF, rows, _PAD), jnp.float32),
            pltpu.SemaphoreType.DMA,
            pltpu.SemaphoreType.DMA,
        ],
        compiler_params=pltpu.CompilerParams(use_tc_tiling_on_sc=False),
    )
    def body(idx_hbm, table_hbm, out_hbm, idx_v, rows_v, gsem0, gsem1):
        cid = lax.axis_index("c")
        sid = lax.axis_index("s")
        wid = sid * 2 + cid
        base = wid * nchunk * _CHUNK
        pltpu.sync_copy(idx_hbm.at[wid], idx_v)
        sems = (gsem0, gsem1)

        def gather_group(g, p, sem):
            for q in range(_K):
                pltpu.async_copy(
                    table_hbm.at[idx_v.at[g * _K + q]],
                    rows_v.at[p].at[pl.ds(q * _CHUNK, _CHUNK)],
                    sem,
                )

        def drain_group(g, p, sem):
            for q in range(_K):
                pltpu.make_async_copy(
                    table_hbm.at[idx_v.at[g * _K + q]],
                    rows_v.at[p].at[pl.ds(q * _CHUNK, _CHUNK)],
                    sem,
                ).wait()

        for p in range(_NBUF):
            gather_group(p, p, sems[p])

        @pl.loop(0, ngroup, step=_NBUF)
        def _(g):
            for p in range(_NBUF):
                cur = g + p
                drain_group(cur, p, sems[p])
                pltpu.sync_copy(
                    rows_v.at[p].at[:, pl.ds(0, _DIM)],
                    out_hbm.at[pl.ds(base + cur * rows, rows), pl.ds(0, _DIM)],
                )
                nxt = cur + _NBUF

                @pl.when(nxt < ngroup)
                def _():
                    gather_group(nxt, p, sems[p])

    return body(idx, table)


@jax.jit
def _embed(token_ids, weight):
    b, s = token_ids.shape
    nchunk = b * s // _NW // _CHUNK
    table = _pad_table(weight.T)
    idx = token_ids.T.reshape(_NW, nchunk, _CHUNK).astype(jnp.int32)
    g = _gather_sc(idx, table, ngroup=nchunk // _K)
    return _untranspose(g, b, s).transpose(2, 0, 1)


def kernel(token_ids, weight):
    return _embed(token_ids, weight)


# speedup vs baseline: 1.1270x; 1.0397x over previous
import functools

import jax
import jax.numpy as jnp
from jax import lax
from jax.experimental import pallas as pl
from jax.experimental.pallas import tpu as pltpu
from jax.experimental.pallas import tpu_sc as plsc

_V = 1_000_000
_DIM = 64
_PAD = 128
_NW = 32
_CHUNK = 128
_K = 2
_NBUF = 2
_ABLK = 4096
_CBLK = 2048


def _pad_table(wt):
    def body(i_ref, o_ref):
        o_ref[:, :_DIM] = i_ref[...].T

    return pl.pallas_call(
        body,
        grid=(pl.cdiv(_V, _ABLK),),
        in_specs=[pl.BlockSpec((_DIM, _ABLK), lambda j: (0, j))],
        out_specs=pl.BlockSpec((_ABLK, _PAD), lambda j: (j, 0)),
        out_shape=jax.ShapeDtypeStruct((_V, _PAD), jnp.float32),
    )(wt)


def _untranspose(g, b, s, s0, stot, y=None):
    nb = b // _CBLK
    gspec = pl.BlockSpec((_CBLK, _PAD), lambda si, j: (si * nb + j, 0))
    ospec = pl.BlockSpec((1, _DIM, _CBLK), lambda si, j: (s0 + si, 0, j))
    oshape = jax.ShapeDtypeStruct((stot, _DIM, b), jnp.float32)

    if y is None:

        def body0(i_ref, o_ref):
            o_ref[...] = i_ref[:, :_DIM].T[None]

        return pl.pallas_call(
            body0, grid=(s, nb), in_specs=[gspec], out_specs=ospec,
            out_shape=oshape,
        )(g)

    def body1(y_ref, i_ref, o_ref):
        o_ref[...] = i_ref[:, :_DIM].T[None]

    return pl.pallas_call(
        body1,
        grid=(s, nb),
        in_specs=[pl.BlockSpec((1, 8, 128), lambda si, j: (0, 0, 0)), gspec],
        out_specs=ospec,
        out_shape=oshape,
        input_output_aliases={0: 0},
    )(y, g)


def _gather_sc(idx, table, *, ngroup):
    nchunk = ngroup * _K
    rows = _K * _CHUNK
    mesh = plsc.VectorSubcoreMesh(
        core_axis_name="c", subcore_axis_name="s", num_cores=2, num_subcores=16
    )

    @functools.partial(
        pl.kernel,
        out_type=jax.ShapeDtypeStruct((_NW * nchunk * _CHUNK, _PAD), jnp.float32),
        mesh=mesh,
        scratch_types=[
            pltpu.VMEM((nchunk, _CHUNK), jnp.int32),
            pltpu.VMEM((_NBUF, rows, _PAD), jnp.float32),
            pltpu.SemaphoreType.DMA,
            pltpu.SemaphoreType.DMA,
        ],
        compiler_params=pltpu.CompilerParams(use_tc_tiling_on_sc=False),
    )
    def body(idx_hbm, table_hbm, out_hbm, idx_v, rows_v, gsem0, gsem1):
        cid = lax.axis_index("c")
        sid = lax.axis_index("s")
        wid = sid * 2 + cid
        base = wid * nchunk * _CHUNK
        pltpu.sync_copy(idx_hbm.at[wid], idx_v)
        sems = (gsem0, gsem1)

        def gather_group(g, p, sem):
            for q in range(_K):
                pltpu.async_copy(
                    table_hbm.at[idx_v.at[g * _K + q]],
                    rows_v.at[p].at[pl.ds(q * _CHUNK, _CHUNK)],
                    sem,
                )

        def drain_group(g, p, sem):
            for q in range(_K):
                pltpu.make_async_copy(
                    table_hbm.at[idx_v.at[g * _K + q]],
                    rows_v.at[p].at[pl.ds(q * _CHUNK, _CHUNK)],
                    sem,
                ).wait()

        for p in range(_NBUF):
            gather_group(p, p, sems[p])

        @pl.loop(0, ngroup, step=_NBUF)
        def _(g):
            for p in range(_NBUF):
                cur = g + p
                drain_group(cur, p, sems[p])
                pltpu.sync_copy(
                    rows_v.at[p].at[:, pl.ds(0, _DIM)],
                    out_hbm.at[pl.ds(base + cur * rows, rows), pl.ds(0, _DIM)],
                )
                nxt = cur + _NBUF

                @pl.when(nxt < ngroup)
                def _():
                    gather_group(nxt, p, sems[p])

    return body(idx, table)


@jax.jit
def _embed(token_ids, weight):
    b, s = token_ids.shape
    half = s // 2
    nchunk = b * half // _NW // _CHUNK
    table = _pad_table(weight.T)
    tt = token_ids.T.astype(jnp.int32)
    idx0 = tt[:half].reshape(_NW, nchunk, _CHUNK)
    idx1 = tt[half:].reshape(_NW, nchunk, _CHUNK)
    g0 = _gather_sc(idx0, table, ngroup=nchunk // _K)
    g1 = _gather_sc(idx1, table, ngroup=nchunk // _K)
    y0 = _untranspose(g0, b, half, 0, s)
    y = _untranspose(g1, b, half, half, s, y=y0)
    return y.transpose(2, 0, 1)


def kernel(token_ids, weight):
    return _embed(token_ids, weight)


# speedup vs baseline: 1.2239x; 1.0859x over previous
import functools

import jax
import jax.numpy as jnp
from jax import lax
from jax.experimental import pallas as pl
from jax.experimental.pallas import tpu as pltpu
from jax.experimental.pallas import tpu_sc as plsc

_V = 1_000_000
_DIM = 64
_PAD = 128
_NW = 32
_CHUNK = 128
_K = 2
_NBUF = 2
_ABLK = 8192
_CBLK = 2048


def _pad_table(wt):
    def body(i_ref, o_ref):
        o_ref[:, :_DIM] = i_ref[...].T

    return pl.pallas_call(
        body,
        grid=(pl.cdiv(_V, _ABLK),),
        in_specs=[pl.BlockSpec((_DIM, _ABLK), lambda j: (0, j))],
        out_specs=pl.BlockSpec((_ABLK, _PAD), lambda j: (j, 0)),
        out_shape=jax.ShapeDtypeStruct((_V, _PAD), jnp.float32),
    )(wt)


def _untranspose(g, b, s, s0, stot, y=None):
    nb = b // _CBLK
    gspec = pl.BlockSpec((_CBLK, _PAD), lambda si, j: (si * nb + j, 0))
    ospec = pl.BlockSpec((1, _DIM, _CBLK), lambda si, j: (s0 + si, 0, j))
    oshape = jax.ShapeDtypeStruct((stot, _DIM, b), jnp.float32)

    if y is None:

        def body0(i_ref, o_ref):
            o_ref[...] = i_ref[:, :_DIM].T[None]

        return pl.pallas_call(
            body0, grid=(s, nb), in_specs=[gspec], out_specs=ospec,
            out_shape=oshape,
        )(g)

    def body1(y_ref, i_ref, o_ref):
        o_ref[...] = i_ref[:, :_DIM].T[None]

    return pl.pallas_call(
        body1,
        grid=(s, nb),
        in_specs=[pl.BlockSpec((1, 8, 128), lambda si, j: (0, 0, 0)), gspec],
        out_specs=ospec,
        out_shape=oshape,
        input_output_aliases={0: 0},
    )(y, g)


def _gather_sc(idx, table, *, ngroup):
    nchunk = ngroup * _K
    rows = _K * _CHUNK
    mesh = plsc.VectorSubcoreMesh(
        core_axis_name="c", subcore_axis_name="s", num_cores=2, num_subcores=16
    )

    @functools.partial(
        pl.kernel,
        out_type=jax.ShapeDtypeStruct((_NW * nchunk * _CHUNK, _PAD), jnp.float32),
        mesh=mesh,
        scratch_types=[
            pltpu.VMEM((nchunk, _CHUNK), jnp.int32),
            pltpu.VMEM((_NBUF, rows, _PAD), jnp.float32),
            pltpu.SemaphoreType.DMA,
            pltpu.SemaphoreType.DMA,
        ],
        compiler_params=pltpu.CompilerParams(use_tc_tiling_on_sc=False),
    )
    def body(idx_hbm, table_hbm, out_hbm, idx_v, rows_v, gsem0, gsem1):
        cid = lax.axis_index("c")
        sid = lax.axis_index("s")
        wid = sid * 2 + cid
        base = wid * nchunk * _CHUNK
        pltpu.sync_copy(idx_hbm.at[wid], idx_v)
        sems = (gsem0, gsem1)

        def gather_group(g, p, sem):
            for q in range(_K):
                pltpu.async_copy(
                    table_hbm.at[idx_v.at[g * _K + q]],
                    rows_v.at[p].at[pl.ds(q * _CHUNK, _CHUNK)],
                    sem,
                )

        def drain_group(g, p, sem):
            for q in range(_K):
                pltpu.make_async_copy(
                    table_hbm.at[idx_v.at[g * _K + q]],
                    rows_v.at[p].at[pl.ds(q * _CHUNK, _CHUNK)],
                    sem,
                ).wait()

        for p in range(_NBUF):
            gather_group(p, p, sems[p])

        @pl.loop(0, ngroup, step=_NBUF)
        def _(g):
            for p in range(_NBUF):
                cur = g + p
                drain_group(cur, p, sems[p])
                pltpu.sync_copy(
                    rows_v.at[p].at[:, pl.ds(0, _DIM)],
                    out_hbm.at[pl.ds(base + cur * rows, rows), pl.ds(0, _DIM)],
                )
                nxt = cur + _NBUF

                @pl.when(nxt < ngroup)
                def _():
                    gather_group(nxt, p, sems[p])

    return body(idx, table)


@jax.jit
def _embed(token_ids, weight):
    b, s = token_ids.shape
    half = s // 2
    nchunk = b * half // _NW // _CHUNK
    table = _pad_table(weight.T)
    tt = token_ids.T.astype(jnp.int32)
    idx0 = tt[:half].reshape(_NW, nchunk, _CHUNK)
    idx1 = tt[half:].reshape(_NW, nchunk, _CHUNK)
    g0 = _gather_sc(idx0, table, ngroup=nchunk // _K)
    g1 = _gather_sc(idx1, table, ngroup=nchunk // _K)
    y0 = _untranspose(g0, b, half, 0, s)
    y = _untranspose(g1, b, half, half, s, y=y0)
    return y.transpose(2, 0, 1)


def kernel(token_ids, weight):
    return _embed(token_ids, weight)


# speedup vs baseline: 1.4101x; 1.1522x over previous
import functools

import jax
import jax.numpy as jnp
from jax import lax
from jax.experimental import pallas as pl
from jax.experimental.pallas import tpu as pltpu
from jax.experimental.pallas import tpu_sc as plsc

_V = 1_000_000
_DIM = 64
_PAD = 128
_NW = 32
_CHUNK = 128
_K = 2
_NBUF = 2
_ABLK = 16384
_CBLK = 4096


def _pad_table(wt):
    def body(i_ref, o_ref):
        o_ref[:, :_DIM] = i_ref[...].T

    return pl.pallas_call(
        body,
        grid=(pl.cdiv(_V, _ABLK),),
        in_specs=[pl.BlockSpec((_DIM, _ABLK), lambda j: (0, j))],
        out_specs=pl.BlockSpec((_ABLK, _PAD), lambda j: (j, 0)),
        out_shape=jax.ShapeDtypeStruct((_V, _PAD), jnp.float32),
    )(wt)


def _untranspose(g, b, s, s0, stot, y=None):
    nb = b // _CBLK
    gspec = pl.BlockSpec((_CBLK, _PAD), lambda si, j: (si * nb + j, 0))
    ospec = pl.BlockSpec((1, _DIM, _CBLK), lambda si, j: (s0 + si, 0, j))
    oshape = jax.ShapeDtypeStruct((stot, _DIM, b), jnp.float32)

    if y is None:

        def body0(i_ref, o_ref):
            o_ref[...] = i_ref[:, :_DIM].T[None]

        return pl.pallas_call(
            body0, grid=(s, nb), in_specs=[gspec], out_specs=ospec,
            out_shape=oshape,
        )(g)

    def body1(y_ref, i_ref, o_ref):
        o_ref[...] = i_ref[:, :_DIM].T[None]

    return pl.pallas_call(
        body1,
        grid=(s, nb),
        in_specs=[pl.BlockSpec((1, 8, 128), lambda si, j: (0, 0, 0)), gspec],
        out_specs=ospec,
        out_shape=oshape,
        input_output_aliases={0: 0},
    )(y, g)


def _gather_sc(idx, table, *, ngroup):
    nchunk = ngroup * _K
    rows = _K * _CHUNK
    mesh = plsc.VectorSubcoreMesh(
        core_axis_name="c", subcore_axis_name="s", num_cores=2, num_subcores=16
    )

    @functools.partial(
        pl.kernel,
        out_type=jax.ShapeDtypeStruct((_NW * nchunk * _CHUNK, _PAD), jnp.float32),
        mesh=mesh,
        scratch_types=[
            pltpu.VMEM((nchunk, _CHUNK), jnp.int32),
            pltpu.VMEM((_NBUF, rows, _PAD), jnp.float32),
            pltpu.SemaphoreType.DMA,
            pltpu.SemaphoreType.DMA,
        ],
        compiler_params=pltpu.CompilerParams(use_tc_tiling_on_sc=False),
    )
    def body(idx_hbm, table_hbm, out_hbm, idx_v, rows_v, gsem0, gsem1):
        cid = lax.axis_index("c")
        sid = lax.axis_index("s")
        wid = sid * 2 + cid
        base = wid * nchunk * _CHUNK
        pltpu.sync_copy(idx_hbm.at[wid], idx_v)
        sems = (gsem0, gsem1)

        def gather_group(g, p, sem):
            for q in range(_K):
                pltpu.async_copy(
                    table_hbm.at[idx_v.at[g * _K + q]],
                    rows_v.at[p].at[pl.ds(q * _CHUNK, _CHUNK)],
                    sem,
                )

        def drain_group(g, p, sem):
            for q in range(_K):
                pltpu.make_async_copy(
                    table_hbm.at[idx_v.at[g * _K + q]],
                    rows_v.at[p].at[pl.ds(q * _CHUNK, _CHUNK)],
                    sem,
                ).wait()

        for p in range(_NBUF):
            gather_group(p, p, sems[p])

        @pl.loop(0, ngroup, step=_NBUF)
        def _(g):
            for p in range(_NBUF):
                cur = g + p
                drain_group(cur, p, sems[p])
                pltpu.sync_copy(
                    rows_v.at[p].at[:, pl.ds(0, _DIM)],
                    out_hbm.at[pl.ds(base + cur * rows, rows), pl.ds(0, _DIM)],
                )
                nxt = cur + _NBUF

                @pl.when(nxt < ngroup)
                def _():
                    gather_group(nxt, p, sems[p])

    return body(idx, table)


@jax.jit
def _embed(token_ids, weight):
    b, s = token_ids.shape
    half = s // 2
    nchunk = b * half // _NW // _CHUNK
    table = _pad_table(weight.T)
    tt = token_ids.T.astype(jnp.int32)
    idx0 = tt[:half].reshape(_NW, nchunk, _CHUNK)
    idx1 = tt[half:].reshape(_NW, nchunk, _CHUNK)
    g0 = _gather_sc(idx0, table, ngroup=nchunk // _K)
    g1 = _gather_sc(idx1, table, ngroup=nchunk // _K)
    y0 = _untranspose(g0, b, half, 0, s)
    y = _untranspose(g1, b, half, half, s, y=y0)
    return y.transpose(2, 0, 1)


def kernel(token_ids, weight):
    return _embed(token_ids, weight)
